# Initial kernel scaffold; baseline (speedup 1.0000x reference)
import jax
import jax.numpy as jnp
from jax.experimental import pallas as pl


def kernel(x, W_fam, fam_emb, W_enc, b_enc, dictionary, bias_correction, ln1_g, ln1_b, Wq, Wk, Wv, Wo, ln2_g, ln2_b, W1, Wg, W2):
    raise NotImplementedError("write your pallas kernel here")


# speedup vs baseline: 3.2953x; 3.2953x over previous
import functools

import jax
import jax.numpy as jnp
from jax.experimental import pallas as pl

_K = 64
_H = 16
_LN_EPS = 1e-5


def _enc_body(x_ref, wfam_ref, femb_ref, fs_ref, basis_ref, resid_ref):
    xx = x_ref[...]
    s = jnp.dot(xx, wfam_ref[...])
    s = s - jnp.max(s, axis=-1, keepdims=True)
    e = jnp.exp(s)
    p = e / jnp.sum(e, axis=-1, keepdims=True)
    fs_ref[...] = p
    b = jnp.dot(p, femb_ref[...])
    basis_ref[...] = b
    resid_ref[...] = xx - b


def _coeff_body(a_ref, w_ref, b_ref, o_ref):
    o_ref[...] = jnp.dot(a_ref[...], w_ref[...]) + b_ref[...]


def _thresh_body(c_ref, t_ref, sp_ref, *, nsteps, denom, n_iter):
    c = c_ref[...]
    lo = jnp.min(c, axis=1, keepdims=True)
    hi = jnp.max(c, axis=1, keepdims=True)

    def it(_, lh):
        lo_, hi_ = lh
        mid = (lo_ + hi_) * 0.5
        cnt = jnp.sum((c >= mid).astype(jnp.float32), axis=1, keepdims=True)
        ge = cnt >= _K
        return (jnp.where(ge, mid, lo_), jnp.where(ge, hi_, mid))

    lo, hi = jax.lax.fori_loop(0, n_iter, it, (lo, hi))
    t_ref[...] = lo
    part = jnp.sum(jnp.where(c >= lo, jnp.abs(c), 0.0)).reshape(1, 1)
    i = pl.program_id(0)

    @pl.when(i == 0)
    def _():
        sp_ref[...] = jnp.zeros((1, 1), jnp.float32)

    sp_ref[...] += part

    @pl.when(i == nsteps - 1)
    def _():
        sp_ref[...] = sp_ref[...] / denom


def _recon_body(c_ref, t_ref, d_ref, basis_ref, bias_ref, o_ref):
    j = pl.program_id(1)
    c = c_ref[...]
    masked = jnp.where(c >= t_ref[...], c, 0.0)

    @pl.when(j == 0)
    def _():
        o_ref[...] = basis_ref[...] + bias_ref[...]

    o_ref[...] += jnp.dot(masked, d_ref[...])


def _ln_body(x_ref, g_ref, b_ref, o_ref, e_ref, *, nsteps, denom):
    xx = x_ref[...]
    mu = jnp.mean(xx, axis=-1, keepdims=True)
    var = jnp.mean((xx - mu) ** 2, axis=-1, keepdims=True)
    o_ref[...] = (xx - mu) / jnp.sqrt(var + _LN_EPS) * g_ref[...] + b_ref[...]
    i = pl.program_id(0)

    @pl.when(i == 0)
    def _():
        e_ref[...] = jnp.zeros((1, 1), jnp.float32)

    e_ref[...] += jnp.sum(xx * xx).reshape(1, 1)

    @pl.when(i == nsteps - 1)
    def _():
        e_ref[...] = e_ref[...] / denom


def _qkv_body(n_ref, wq_ref, wk_ref, wv_ref, q_ref, k_ref, v_ref):
    n = n_ref[...]
    q_ref[...] = jnp.dot(n, wq_ref[...])
    k_ref[...] = jnp.dot(n, wk_ref[...])
    v_ref[...] = jnp.dot(n, wv_ref[...])


def _attn_body(q_ref, k_ref, v_ref, w_ref, ctx_ref, *, scale):
    q = q_ref[0, 0]
    k = k_ref[0, 0]
    v = v_ref[0, 0]
    s = jax.lax.dot_general(q, k, (((1,), (1,)), ((), ()))) * scale
    s = s - jnp.max(s, axis=-1, keepdims=True)
    e = jnp.exp(s)
    w = e / jnp.sum(e, axis=-1, keepdims=True)
    w_ref[0, 0] = w
    ctx_ref[0, 0] = jnp.dot(w, v)


def _proj_add_body(c_ref, wo_ref, x_ref, o_ref):
    o_ref[...] = jnp.dot(c_ref[...], wo_ref[...]) + x_ref[...]


def _ffn_body(n_ref, wg_ref, w1_ref, w2_ref, x_ref, o_ref):
    j = pl.program_id(1)
    n = n_ref[...]
    h = jax.nn.sigmoid(jnp.dot(n, wg_ref[...])) * jax.nn.gelu(jnp.dot(n, w1_ref[...]))

    @pl.when(j == 0)
    def _():
        o_ref[...] = x_ref[...]

    o_ref[...] += jnp.dot(h, w2_ref[...])


def kernel(x, W_fam, fam_emb, W_enc, b_enc, dictionary, bias_correction,
           ln1_g, ln1_b, Wq, Wk, Wv, Wo, ln2_g, ln2_b, W1, Wg, W2):
    B, T, D = x.shape
    F = W_fam.shape[1]
    M = W_enc.shape[1]
    DFF = W1.shape[1]
    N = B * T
    H = _H
    dh = D // H

    R = 256
    nt = N // R
    MT = 512
    nmt = M // MT
    FT = 512
    nft = DFF // FT
    TQ = 256
    f32 = jnp.float32

    xf = x.reshape(N, D)

    fs, basis, resid = pl.pallas_call(
        _enc_body,
        grid=(nt,),
        in_specs=[
            pl.BlockSpec((R, D), lambda i: (i, 0)),
            pl.BlockSpec((D, F), lambda i: (0, 0)),
            pl.BlockSpec((F, D), lambda i: (0, 0)),
        ],
        out_specs=[
            pl.BlockSpec((R, F), lambda i: (i, 0)),
            pl.BlockSpec((R, D), lambda i: (i, 0)),
            pl.BlockSpec((R, D), lambda i: (i, 0)),
        ],
        out_shape=[
            jax.ShapeDtypeStruct((N, F), f32),
            jax.ShapeDtypeStruct((N, D), f32),
            jax.ShapeDtypeStruct((N, D), f32),
        ],
    )(xf, W_fam, fam_emb)

    coeffs = pl.pallas_call(
        _coeff_body,
        grid=(nt, nmt),
        in_specs=[
            pl.BlockSpec((R, D), lambda i, j: (i, 0)),
            pl.BlockSpec((D, MT), lambda i, j: (0, j)),
            pl.BlockSpec((1, MT), lambda i, j: (0, j)),
        ],
        out_specs=pl.BlockSpec((R, MT), lambda i, j: (i, j)),
        out_shape=jax.ShapeDtypeStruct((N, M), f32),
    )(resid, W_enc, b_enc.reshape(1, M))

    thresh, sp = pl.pallas_call(
        functools.partial(_thresh_body, nsteps=nt, denom=float(N * M), n_iter=48),
        grid=(nt,),
        in_specs=[pl.BlockSpec((R, M), lambda i: (i, 0))],
        out_specs=[
            pl.BlockSpec((R, 1), lambda i: (i, 0)),
            pl.BlockSpec((1, 1), lambda i: (0, 0)),
        ],
        out_shape=[
            jax.ShapeDtypeStruct((N, 1), f32),
            jax.ShapeDtypeStruct((1, 1), f32),
        ],
    )(coeffs)

    xrec = pl.pallas_call(
        _recon_body,
        grid=(nt, nmt),
        in_specs=[
            pl.BlockSpec((R, MT), lambda i, j: (i, j)),
            pl.BlockSpec((R, 1), lambda i, j: (i, 0)),
            pl.BlockSpec((MT, D), lambda i, j: (j, 0)),
            pl.BlockSpec((R, D), lambda i, j: (i, 0)),
            pl.BlockSpec((1, D), lambda i, j: (0, 0)),
        ],
        out_specs=pl.BlockSpec((R, D), lambda i, j: (i, 0)),
        out_shape=jax.ShapeDtypeStruct((N, D), f32),
    )(coeffs, thresh, dictionary, basis, bias_correction.reshape(1, D))

    normed, e1 = pl.pallas_call(
        functools.partial(_ln_body, nsteps=nt, denom=float(N * D)),
        grid=(nt,),
        in_specs=[
            pl.BlockSpec((R, D), lambda i: (i, 0)),
            pl.BlockSpec((1, D), lambda i: (0, 0)),
            pl.BlockSpec((1, D), lambda i: (0, 0)),
        ],
        out_specs=[
            pl.BlockSpec((R, D), lambda i: (i, 0)),
            pl.BlockSpec((1, 1), lambda i: (0, 0)),
        ],
        out_shape=[
            jax.ShapeDtypeStruct((N, D), f32),
            jax.ShapeDtypeStruct((1, 1), f32),
        ],
    )(xrec, ln1_g.reshape(1, D), ln1_b.reshape(1, D))

    qf, kf, vf = pl.pallas_call(
        _qkv_body,
        grid=(nt,),
        in_specs=[
            pl.BlockSpec((R, D), lambda i: (i, 0)),
            pl.BlockSpec((D, D), lambda i: (0, 0)),
            pl.BlockSpec((D, D), lambda i: (0, 0)),
            pl.BlockSpec((D, D), lambda i: (0, 0)),
        ],
        out_specs=[
            pl.BlockSpec((R, D), lambda i: (i, 0)),
            pl.BlockSpec((R, D), lambda i: (i, 0)),
            pl.BlockSpec((R, D), lambda i: (i, 0)),
        ],
        out_shape=[
            jax.ShapeDtypeStruct((N, D), f32),
            jax.ShapeDtypeStruct((N, D), f32),
            jax.ShapeDtypeStruct((N, D), f32),
        ],
    )(normed, Wq, Wk, Wv)

    q4 = qf.reshape(B, T, H, dh).transpose(0, 2, 1, 3)
    k4 = kf.reshape(B, T, H, dh).transpose(0, 2, 1, 3)
    v4 = vf.reshape(B, T, H, dh).transpose(0, 2, 1, 3)

    attn_w, ctx4 = pl.pallas_call(
        functools.partial(_attn_body, scale=1.0 / float(dh) ** 0.5),
        grid=(B, H, T // TQ),
        in_specs=[
            pl.BlockSpec((1, 1, TQ, dh), lambda b, h, i: (b, h, i, 0)),
            pl.BlockSpec((1, 1, T, dh), lambda b, h, i: (b, h, 0, 0)),
            pl.BlockSpec((1, 1, T, dh), lambda b, h, i: (b, h, 0, 0)),
        ],
        out_specs=[
            pl.BlockSpec((1, 1, TQ, T), lambda b, h, i: (b, h, i, 0)),
            pl.BlockSpec((1, 1, TQ, dh), lambda b, h, i: (b, h, i, 0)),
        ],
        out_shape=[
            jax.ShapeDtypeStruct((B, H, T, T), f32),
            jax.ShapeDtypeStruct((B, H, T, dh), f32),
        ],
    )(q4, k4, v4)

    ctx = ctx4.transpose(0, 2, 1, 3).reshape(N, D)

    x1 = pl.pallas_call(
        _proj_add_body,
        grid=(nt,),
        in_specs=[
            pl.BlockSpec((R, D), lambda i: (i, 0)),
            pl.BlockSpec((D, D), lambda i: (0, 0)),
            pl.BlockSpec((R, D), lambda i: (i, 0)),
        ],
        out_specs=pl.BlockSpec((R, D), lambda i: (i, 0)),
        out_shape=jax.ShapeDtypeStruct((N, D), f32),
    )(ctx, Wo, xf)

    normed2, e2 = pl.pallas_call(
        functools.partial(_ln_body, nsteps=nt, denom=float(N * D)),
        grid=(nt,),
        in_specs=[
            pl.BlockSpec((R, D), lambda i: (i, 0)),
            pl.BlockSpec((1, D), lambda i: (0, 0)),
            pl.BlockSpec((1, D), lambda i: (0, 0)),
        ],
        out_specs=[
            pl.BlockSpec((R, D), lambda i: (i, 0)),
            pl.BlockSpec((1, 1), lambda i: (0, 0)),
        ],
        out_shape=[
            jax.ShapeDtypeStruct((N, D), f32),
            jax.ShapeDtypeStruct((1, 1), f32),
        ],
    )(x1, ln2_g.reshape(1, D), ln2_b.reshape(1, D))

    xout = pl.pallas_call(
        _ffn_body,
        grid=(nt, nft),
        in_specs=[
            pl.BlockSpec((R, D), lambda i, j: (i, 0)),
            pl.BlockSpec((D, FT), lambda i, j: (0, j)),
            pl.BlockSpec((D, FT), lambda i, j: (0, j)),
            pl.BlockSpec((FT, D), lambda i, j: (j, 0)),
            pl.BlockSpec((R, D), lambda i, j: (i, 0)),
        ],
        out_specs=pl.BlockSpec((R, D), lambda i, j: (i, 0)),
        out_shape=jax.ShapeDtypeStruct((N, D), f32),
    )(normed2, Wg, W1, W2, x1)

    return (
        xout.reshape(B, T, D),
        attn_w,
        fs.reshape(B, T, F),
        sp.reshape(()),
        e1.reshape(()),
        e2.reshape(()),
    )


# speedup vs baseline: 4.9380x; 1.4985x over previous
import functools

import jax
import jax.numpy as jnp
from jax.experimental import pallas as pl

_K = 64
_H = 16
_LN_EPS = 1e-5


def _ln(xx, g, b):
    mu = jnp.mean(xx, axis=-1, keepdims=True)
    var = jnp.mean((xx - mu) ** 2, axis=-1, keepdims=True)
    return (xx - mu) / jnp.sqrt(var + _LN_EPS) * g + b


def _stageA_body(x_ref, wfam_ref, femb_ref, wenc_ref, benc_ref,
                 fs_ref, basis_ref, mc_ref, sp_ref, *, nsteps, denom, n_iter):
    xx = x_ref[...]
    s = jnp.dot(xx, wfam_ref[...])
    s = s - jnp.max(s, axis=-1, keepdims=True)
    e = jnp.exp(s)
    p = e / jnp.sum(e, axis=-1, keepdims=True)
    fs_ref[...] = p
    basis = jnp.dot(p, femb_ref[...])
    basis_ref[...] = basis
    c = jnp.dot(xx - basis, wenc_ref[...]) + benc_ref[...]

    lo = jnp.min(c, axis=1, keepdims=True)
    hi = jnp.max(c, axis=1, keepdims=True)

    def it(_, lh):
        lo_, hi_ = lh
        mid = (lo_ + hi_) * 0.5
        cnt = jnp.sum((c >= mid).astype(jnp.float32), axis=1, keepdims=True)
        ge = cnt >= _K
        return (jnp.where(ge, mid, lo_), jnp.where(ge, hi_, mid))

    lo, hi = jax.lax.fori_loop(0, n_iter, it, (lo, hi))
    masked = jnp.where(c >= lo, c, 0.0)
    mc_ref[...] = masked

    i = pl.program_id(0)

    @pl.when(i == 0)
    def _():
        sp_ref[...] = jnp.zeros((1, 1), jnp.float32)

    sp_ref[...] += jnp.sum(jnp.abs(masked)).reshape(1, 1)

    @pl.when(i == nsteps - 1)
    def _():
        sp_ref[...] = sp_ref[...] / denom


def _stageB_body(mc_ref, d_ref, basis_ref, bias_ref, g_ref, b_ref,
                 n_ref, e_ref, *, nmt, nsteps, denom):
    i = pl.program_id(0)
    j = pl.program_id(1)

    @pl.when(j == 0)
    def _():
        n_ref[...] = basis_ref[...] + bias_ref[...]

    n_ref[...] += jnp.dot(mc_ref[...], d_ref[...])

    @pl.when(j == nmt - 1)
    def _():
        xr = n_ref[...]

        @pl.when(i == 0)
        def _():
            e_ref[...] = jnp.zeros((1, 1), jnp.float32)

        e_ref[...] += jnp.sum(xr * xr).reshape(1, 1)

        @pl.when(i == nsteps - 1)
        def _():
            e_ref[...] = e_ref[...] / denom

        n_ref[...] = _ln(xr, g_ref[...], b_ref[...])


def _qkv_body(n_ref, wq_ref, wk_ref, wv_ref, q_ref, k_ref, v_ref):
    n = n_ref[...]
    q_ref[...] = jnp.dot(n, wq_ref[...])
    k_ref[...] = jnp.dot(n, wk_ref[...])
    v_ref[...] = jnp.dot(n, wv_ref[...])


def _attn_body(q_ref, k_ref, v_ref, w_ref, ctx_ref, *, scale, dh):
    q2 = q_ref[0]
    k2 = k_ref[0]
    v2 = v_ref[0]
    outs = []
    for h2 in range(2):
        qh = q2[:, h2 * dh:(h2 + 1) * dh]
        kh = k2[:, h2 * dh:(h2 + 1) * dh]
        vh = v2[:, h2 * dh:(h2 + 1) * dh]
        s = jax.lax.dot_general(qh, kh, (((1,), (1,)), ((), ()))) * scale
        s = s - jnp.max(s, axis=-1, keepdims=True)
        e = jnp.exp(s)
        w = e / jnp.sum(e, axis=-1, keepdims=True)
        w_ref[0, h2] = w
        outs.append(jnp.dot(w, vh))
    ctx_ref[0] = jnp.concatenate(outs, axis=1)


def _stageE_body(c_ref, wo_ref, x_ref, g_ref, b_ref, x1_ref, n2_ref, e_ref,
                 *, nsteps, denom):
    x1 = jnp.dot(c_ref[...], wo_ref[...]) + x_ref[...]
    x1_ref[...] = x1
    i = pl.program_id(0)

    @pl.when(i == 0)
    def _():
        e_ref[...] = jnp.zeros((1, 1), jnp.float32)

    e_ref[...] += jnp.sum(x1 * x1).reshape(1, 1)

    @pl.when(i == nsteps - 1)
    def _():
        e_ref[...] = e_ref[...] / denom

    n2_ref[...] = _ln(x1, g_ref[...], b_ref[...])


def _ffn_body(n_ref, wg_ref, w1_ref, w2_ref, x_ref, o_ref):
    j = pl.program_id(1)
    n = n_ref[...]
    h = jax.nn.sigmoid(jnp.dot(n, wg_ref[...])) * jax.nn.gelu(jnp.dot(n, w1_ref[...]))

    @pl.when(j == 0)
    def _():
        o_ref[...] = x_ref[...]

    o_ref[...] += jnp.dot(h, w2_ref[...])


def kernel(x, W_fam, fam_emb, W_enc, b_enc, dictionary, bias_correction,
           ln1_g, ln1_b, Wq, Wk, Wv, Wo, ln2_g, ln2_b, W1, Wg, W2):
    B, T, D = x.shape
    F = W_fam.shape[1]
    M = W_enc.shape[1]
    DFF = W1.shape[1]
    N = B * T
    H = _H
    dh = D // H

    R = 256
    nt = N // R
    MT = 512
    nmt = M // MT
    FT = 512
    nft = DFF // FT
    TQ = 256
    f32 = jnp.float32

    xf = x.reshape(N, D)

    fs, basis, masked, sp = pl.pallas_call(
        functools.partial(_stageA_body, nsteps=nt, denom=float(N * M), n_iter=28),
        grid=(nt,),
        in_specs=[
            pl.BlockSpec((R, D), lambda i: (i, 0)),
            pl.BlockSpec((D, F), lambda i: (0, 0)),
            pl.BlockSpec((F, D), lambda i: (0, 0)),
            pl.BlockSpec((D, M), lambda i: (0, 0)),
            pl.BlockSpec((1, M), lambda i: (0, 0)),
        ],
        out_specs=[
            pl.BlockSpec((R, F), lambda i: (i, 0)),
            pl.BlockSpec((R, D), lambda i: (i, 0)),
            pl.BlockSpec((R, M), lambda i: (i, 0)),
            pl.BlockSpec((1, 1), lambda i: (0, 0)),
        ],
        out_shape=[
            jax.ShapeDtypeStruct((N, F), f32),
            jax.ShapeDtypeStruct((N, D), f32),
            jax.ShapeDtypeStruct((N, M), f32),
            jax.ShapeDtypeStruct((1, 1), f32),
        ],
    )(xf, W_fam, fam_emb, W_enc, b_enc.reshape(1, M))

    normed, e1 = pl.pallas_call(
        functools.partial(_stageB_body, nmt=nmt, nsteps=nt, denom=float(N * D)),
        grid=(nt, nmt),
        in_specs=[
            pl.BlockSpec((R, MT), lambda i, j: (i, j)),
            pl.BlockSpec((MT, D), lambda i, j: (j, 0)),
            pl.BlockSpec((R, D), lambda i, j: (i, 0)),
            pl.BlockSpec((1, D), lambda i, j: (0, 0)),
            pl.BlockSpec((1, D), lambda i, j: (0, 0)),
            pl.BlockSpec((1, D), lambda i, j: (0, 0)),
        ],
        out_specs=[
            pl.BlockSpec((R, D), lambda i, j: (i, 0)),
            pl.BlockSpec((1, 1), lambda i, j: (0, 0)),
        ],
        out_shape=[
            jax.ShapeDtypeStruct((N, D), f32),
            jax.ShapeDtypeStruct((1, 1), f32),
        ],
    )(masked, dictionary, basis, bias_correction.reshape(1, D),
      ln1_g.reshape(1, D), ln1_b.reshape(1, D))

    qf, kf, vf = pl.pallas_call(
        _qkv_body,
        grid=(nt,),
        in_specs=[
            pl.BlockSpec((R, D), lambda i: (i, 0)),
            pl.BlockSpec((D, D), lambda i: (0, 0)),
            pl.BlockSpec((D, D), lambda i: (0, 0)),
            pl.BlockSpec((D, D), lambda i: (0, 0)),
        ],
        out_specs=[
            pl.BlockSpec((R, D), lambda i: (i, 0)),
            pl.BlockSpec((R, D), lambda i: (i, 0)),
            pl.BlockSpec((R, D), lambda i: (i, 0)),
        ],
        out_shape=[
            jax.ShapeDtypeStruct((N, D), f32),
            jax.ShapeDtypeStruct((N, D), f32),
            jax.ShapeDtypeStruct((N, D), f32),
        ],
    )(normed, Wq, Wk, Wv)

    q3 = qf.reshape(B, T, D)
    k3 = kf.reshape(B, T, D)
    v3 = vf.reshape(B, T, D)

    attn_w, ctx3 = pl.pallas_call(
        functools.partial(_attn_body, scale=1.0 / float(dh) ** 0.5, dh=dh),
        grid=(B, H // 2, T // TQ),
        in_specs=[
            pl.BlockSpec((1, TQ, 2 * dh), lambda b, hp, i: (b, i, hp)),
            pl.BlockSpec((1, T, 2 * dh), lambda b, hp, i: (b, 0, hp)),
            pl.BlockSpec((1, T, 2 * dh), lambda b, hp, i: (b, 0, hp)),
        ],
        out_specs=[
            pl.BlockSpec((1, 2, TQ, T), lambda b, hp, i: (b, hp, i, 0)),
            pl.BlockSpec((1, TQ, 2 * dh), lambda b, hp, i: (b, i, hp)),
        ],
        out_shape=[
            jax.ShapeDtypeStruct((B, H, T, T), f32),
            jax.ShapeDtypeStruct((B, T, D), f32),
        ],
    )(q3, k3, v3)

    ctx = ctx3.reshape(N, D)

    x1, normed2, e2 = pl.pallas_call(
        functools.partial(_stageE_body, nsteps=nt, denom=float(N * D)),
        grid=(nt,),
        in_specs=[
            pl.BlockSpec((R, D), lambda i: (i, 0)),
            pl.BlockSpec((D, D), lambda i: (0, 0)),
            pl.BlockSpec((R, D), lambda i: (i, 0)),
            pl.BlockSpec((1, D), lambda i: (0, 0)),
            pl.BlockSpec((1, D), lambda i: (0, 0)),
        ],
        out_specs=[
            pl.BlockSpec((R, D), lambda i: (i, 0)),
            pl.BlockSpec((R, D), lambda i: (i, 0)),
            pl.BlockSpec((1, 1), lambda i: (0, 0)),
        ],
        out_shape=[
            jax.ShapeDtypeStruct((N, D), f32),
            jax.ShapeDtypeStruct((N, D), f32),
            jax.ShapeDtypeStruct((1, 1), f32),
        ],
    )(ctx, Wo, xf, ln2_g.reshape(1, D), ln2_b.reshape(1, D))

    xout = pl.pallas_call(
        _ffn_body,
        grid=(nt, nft),
        in_specs=[
            pl.BlockSpec((R, D), lambda i, j: (i, 0)),
            pl.BlockSpec((D, FT), lambda i, j: (0, j)),
            pl.BlockSpec((D, FT), lambda i, j: (0, j)),
            pl.BlockSpec((FT, D), lambda i, j: (j, 0)),
            pl.BlockSpec((R, D), lambda i, j: (i, 0)),
        ],
        out_specs=pl.BlockSpec((R, D), lambda i, j: (i, 0)),
        out_shape=jax.ShapeDtypeStruct((N, D), f32),
    )(normed2, Wg, W1, W2, x1)

    return (
        xout.reshape(B, T, D),
        attn_w,
        fs.reshape(B, T, F),
        sp.reshape(()),
        e1.reshape(()),
        e2.reshape(()),
    )


# speedup vs baseline: 4.9396x; 1.0003x over previous
import functools

import jax
import jax.numpy as jnp
from jax.experimental import pallas as pl
from jax.experimental.pallas import tpu as pltpu

_K = 64
_H = 16
_LN_EPS = 1e-5


def _ln(xx, g, b):
    mu = jnp.mean(xx, axis=-1, keepdims=True)
    var = jnp.mean((xx - mu) ** 2, axis=-1, keepdims=True)
    return (xx - mu) / jnp.sqrt(var + _LN_EPS) * g + b


def _softmax_last(s):
    s = s - jnp.max(s, axis=-1, keepdims=True)
    e = jnp.exp(s)
    return e / jnp.sum(e, axis=-1, keepdims=True)


def _stageA_body(x_ref, wfam_ref, femb_ref, wenc_ref, benc_ref,
                 fs_ref, basis_ref, mc_ref, sp_ref, *, nsteps, denom, n_iter):
    xx = x_ref[...]
    s = jnp.dot(xx, wfam_ref[...])
    s = s - jnp.max(s, axis=-1, keepdims=True)
    e = jnp.exp(s)
    p = e / jnp.sum(e, axis=-1, keepdims=True)
    fs_ref[...] = p
    basis = jnp.dot(p, femb_ref[...])
    basis_ref[...] = basis
    c = jnp.dot(xx - basis, wenc_ref[...]) + benc_ref[...]

    lo = jnp.min(c, axis=1, keepdims=True)
    hi = jnp.max(c, axis=1, keepdims=True)

    def it(_, lh):
        lo_, hi_ = lh
        mid = (lo_ + hi_) * 0.5
        cnt = jnp.sum((c >= mid).astype(jnp.float32), axis=1, keepdims=True)
        ge = cnt >= _K
        return (jnp.where(ge, mid, lo_), jnp.where(ge, hi_, mid))

    lo, hi = jax.lax.fori_loop(0, n_iter, it, (lo, hi))
    masked = jnp.where(c >= lo, c, 0.0)
    mc_ref[...] = masked

    i = pl.program_id(0)

    @pl.when(i == 0)
    def _():
        sp_ref[...] = jnp.zeros((1, 1), jnp.float32)

    sp_ref[...] += jnp.sum(jnp.abs(masked)).reshape(1, 1)

    @pl.when(i == nsteps - 1)
    def _():
        sp_ref[...] = sp_ref[...] / denom


def _stageB_body(mc_ref, d_ref, basis_ref, bias_ref, g_ref, b_ref,
                 wq_ref, wk_ref, wv_ref,
                 q_ref, k_ref, v_ref, e_ref, acc_ref, *, nmt, nsteps, denom):
    i = pl.program_id(0)
    j = pl.program_id(1)

    @pl.when(j == 0)
    def _():
        acc_ref[...] = basis_ref[...] + bias_ref[...]

    acc_ref[...] += jnp.dot(mc_ref[...], d_ref[...])

    @pl.when(j == nmt - 1)
    def _():
        xr = acc_ref[...]

        @pl.when(i == 0)
        def _():
            e_ref[...] = jnp.zeros((1, 1), jnp.float32)

        e_ref[...] += jnp.sum(xr * xr).reshape(1, 1)

        @pl.when(i == nsteps - 1)
        def _():
            e_ref[...] = e_ref[...] / denom

        n = _ln(xr, g_ref[...], b_ref[...])
        q_ref[...] = jnp.dot(n, wq_ref[...])
        k_ref[...] = jnp.dot(n, wk_ref[...])
        v_ref[...] = jnp.dot(n, wv_ref[...])


def _attn_body(q_ref, k_ref, v_ref, w_ref, ctx_ref, kst_ref, vst_ref,
               *, scale, dh, t_len):
    i = pl.program_id(2)

    @pl.when(i == 0)
    def _():
        k2 = k_ref[0]
        v2 = v_ref[0]
        z = jnp.zeros((t_len, dh), jnp.float32)
        kst_ref[...] = jnp.concatenate([
            jnp.concatenate([k2[:, :dh], z], axis=1),
            jnp.concatenate([z, k2[:, dh:]], axis=1)], axis=0)
        vst_ref[...] = jnp.concatenate([
            jnp.concatenate([v2[:, :dh], z], axis=1),
            jnp.concatenate([z, v2[:, dh:]], axis=1)], axis=0)

    q2 = q_ref[0]
    s_cat = jax.lax.dot_general(q2, kst_ref[...], (((1,), (1,)), ((), ()))) * scale
    w0 = _softmax_last(s_cat[:, :t_len])
    w1 = _softmax_last(s_cat[:, t_len:])
    w_ref[0, 0] = w0
    w_ref[0, 1] = w1
    w_cat = jnp.concatenate([w0, w1], axis=1)
    ctx_ref[0] = jnp.dot(w_cat, vst_ref[...])


def _stageE_body(c_ref, wo_ref, x_ref, g_ref, b_ref, x1_ref, n2_ref, e_ref,
                 *, nsteps, denom):
    x1 = jnp.dot(c_ref[...], wo_ref[...]) + x_ref[...]
    x1_ref[...] = x1
    i = pl.program_id(0)

    @pl.when(i == 0)
    def _():
        e_ref[...] = jnp.zeros((1, 1), jnp.float32)

    e_ref[...] += jnp.sum(x1 * x1).reshape(1, 1)

    @pl.when(i == nsteps - 1)
    def _():
        e_ref[...] = e_ref[...] / denom

    n2_ref[...] = _ln(x1, g_ref[...], b_ref[...])


def _ffn_body(n_ref, wg_ref, w1_ref, w2_ref, x_ref, o_ref):
    j = pl.program_id(1)
    n = n_ref[...]
    h = jax.nn.sigmoid(jnp.dot(n, wg_ref[...])) * jax.nn.gelu(jnp.dot(n, w1_ref[...]))

    @pl.when(j == 0)
    def _():
        o_ref[...] = x_ref[...]

    o_ref[...] += jnp.dot(h, w2_ref[...])


def kernel(x, W_fam, fam_emb, W_enc, b_enc, dictionary, bias_correction,
           ln1_g, ln1_b, Wq, Wk, Wv, Wo, ln2_g, ln2_b, W1, Wg, W2):
    B, T, D = x.shape
    F = W_fam.shape[1]
    M = W_enc.shape[1]
    DFF = W1.shape[1]
    N = B * T
    H = _H
    dh = D // H

    R = 256
    nt = N // R
    MT = 512
    nmt = M // MT
    FT = 512
    nft = DFF // FT
    TQ = 256
    f32 = jnp.float32

    xf = x.reshape(N, D)

    fs, basis, masked, sp = pl.pallas_call(
        functools.partial(_stageA_body, nsteps=nt, denom=float(N * M), n_iter=28),
        grid=(nt,),
        in_specs=[
            pl.BlockSpec((R, D), lambda i: (i, 0)),
            pl.BlockSpec((D, F), lambda i: (0, 0)),
            pl.BlockSpec((F, D), lambda i: (0, 0)),
            pl.BlockSpec((D, M), lambda i: (0, 0)),
            pl.BlockSpec((1, M), lambda i: (0, 0)),
        ],
        out_specs=[
            pl.BlockSpec((R, F), lambda i: (i, 0)),
            pl.BlockSpec((R, D), lambda i: (i, 0)),
            pl.BlockSpec((R, M), lambda i: (i, 0)),
            pl.BlockSpec((1, 1), lambda i: (0, 0)),
        ],
        out_shape=[
            jax.ShapeDtypeStruct((N, F), f32),
            jax.ShapeDtypeStruct((N, D), f32),
            jax.ShapeDtypeStruct((N, M), f32),
            jax.ShapeDtypeStruct((1, 1), f32),
        ],
    )(xf, W_fam, fam_emb, W_enc, b_enc.reshape(1, M))

    qf, kf, vf, e1 = pl.pallas_call(
        functools.partial(_stageB_body, nmt=nmt, nsteps=nt, denom=float(N * D)),
        grid=(nt, nmt),
        in_specs=[
            pl.BlockSpec((R, MT), lambda i, j: (i, j)),
            pl.BlockSpec((MT, D), lambda i, j: (j, 0)),
            pl.BlockSpec((R, D), lambda i, j: (i, 0)),
            pl.BlockSpec((1, D), lambda i, j: (0, 0)),
            pl.BlockSpec((1, D), lambda i, j: (0, 0)),
            pl.BlockSpec((1, D), lambda i, j: (0, 0)),
            pl.BlockSpec((D, D), lambda i, j: (0, 0)),
            pl.BlockSpec((D, D), lambda i, j: (0, 0)),
            pl.BlockSpec((D, D), lambda i, j: (0, 0)),
        ],
        out_specs=[
            pl.BlockSpec((R, D), lambda i, j: (i, 0)),
            pl.BlockSpec((R, D), lambda i, j: (i, 0)),
            pl.BlockSpec((R, D), lambda i, j: (i, 0)),
            pl.BlockSpec((1, 1), lambda i, j: (0, 0)),
        ],
        out_shape=[
            jax.ShapeDtypeStruct((N, D), f32),
            jax.ShapeDtypeStruct((N, D), f32),
            jax.ShapeDtypeStruct((N, D), f32),
            jax.ShapeDtypeStruct((1, 1), f32),
        ],
        scratch_shapes=[pltpu.VMEM((R, D), f32)],
    )(masked, dictionary, basis, bias_correction.reshape(1, D),
      ln1_g.reshape(1, D), ln1_b.reshape(1, D), Wq, Wk, Wv)

    q3 = qf.reshape(B, T, D)
    k3 = kf.reshape(B, T, D)
    v3 = vf.reshape(B, T, D)

    attn_w, ctx3 = pl.pallas_call(
        functools.partial(_attn_body, scale=1.0 / float(dh) ** 0.5, dh=dh, t_len=T),
        grid=(B, H // 2, T // TQ),
        in_specs=[
            pl.BlockSpec((1, TQ, 2 * dh), lambda b, hp, i: (b, i, hp)),
            pl.BlockSpec((1, T, 2 * dh), lambda b, hp, i: (b, 0, hp)),
            pl.BlockSpec((1, T, 2 * dh), lambda b, hp, i: (b, 0, hp)),
        ],
        out_specs=[
            pl.BlockSpec((1, 2, TQ, T), lambda b, hp, i: (b, hp, i, 0)),
            pl.BlockSpec((1, TQ, 2 * dh), lambda b, hp, i: (b, i, hp)),
        ],
        out_shape=[
            jax.ShapeDtypeStruct((B, H, T, T), f32),
            jax.ShapeDtypeStruct((B, T, D), f32),
        ],
        scratch_shapes=[
            pltpu.VMEM((2 * T, 2 * dh), f32),
            pltpu.VMEM((2 * T, 2 * dh), f32),
        ],
    )(q3, k3, v3)

    ctx = ctx3.reshape(N, D)

    x1, normed2, e2 = pl.pallas_call(
        functools.partial(_stageE_body, nsteps=nt, denom=float(N * D)),
        grid=(nt,),
        in_specs=[
            pl.BlockSpec((R, D), lambda i: (i, 0)),
            pl.BlockSpec((D, D), lambda i: (0, 0)),
            pl.BlockSpec((R, D), lambda i: (i, 0)),
            pl.BlockSpec((1, D), lambda i: (0, 0)),
            pl.BlockSpec((1, D), lambda i: (0, 0)),
        ],
        out_specs=[
            pl.BlockSpec((R, D), lambda i: (i, 0)),
            pl.BlockSpec((R, D), lambda i: (i, 0)),
            pl.BlockSpec((1, 1), lambda i: (0, 0)),
        ],
        out_shape=[
            jax.ShapeDtypeStruct((N, D), f32),
            jax.ShapeDtypeStruct((N, D), f32),
            jax.ShapeDtypeStruct((1, 1), f32),
        ],
    )(ctx, Wo, xf, ln2_g.reshape(1, D), ln2_b.reshape(1, D))

    xout = pl.pallas_call(
        _ffn_body,
        grid=(nt, nft),
        in_specs=[
            pl.BlockSpec((R, D), lambda i, j: (i, 0)),
            pl.BlockSpec((D, FT), lambda i, j: (0, j)),
            pl.BlockSpec((D, FT), lambda i, j: (0, j)),
            pl.BlockSpec((FT, D), lambda i, j: (j, 0)),
            pl.BlockSpec((R, D), lambda i, j: (i, 0)),
        ],
        out_specs=pl.BlockSpec((R, D), lambda i, j: (i, 0)),
        out_shape=jax.ShapeDtypeStruct((N, D), f32),
    )(normed2, Wg, W1, W2, x1)

    return (
        xout.reshape(B, T, D),
        attn_w,
        fs.reshape(B, T, F),
        sp.reshape(()),
        e1.reshape(()),
        e2.reshape(()),
    )


# speedup vs baseline: 5.1495x; 1.0425x over previous
import functools

import jax
import jax.numpy as jnp
from jax.experimental import pallas as pl
from jax.experimental.pallas import tpu as pltpu

_K = 64
_H = 16
_LN_EPS = 1e-5


def _ln(xx, g, b):
    mu = jnp.mean(xx, axis=-1, keepdims=True)
    var = jnp.mean((xx - mu) ** 2, axis=-1, keepdims=True)
    return (xx - mu) / jnp.sqrt(var + _LN_EPS) * g + b


def _softmax_last(s):
    s = s - jnp.max(s, axis=-1, keepdims=True)
    e = jnp.exp(s)
    return e / jnp.sum(e, axis=-1, keepdims=True)


def _stageA_body(x_ref, wfam_ref, femb_ref, wenc_ref, benc_ref,
                 fs_ref, basis_ref, mc_ref, sp_ref, *, nsteps, denom, n_iter):
    xx = x_ref[...]
    s = jnp.dot(xx, wfam_ref[...])
    s = s - jnp.max(s, axis=-1, keepdims=True)
    e = jnp.exp(s)
    p = e / jnp.sum(e, axis=-1, keepdims=True)
    fs_ref[...] = p
    basis = jnp.dot(p, femb_ref[...])
    basis_ref[...] = basis
    c = jnp.dot(xx - basis, wenc_ref[...]) + benc_ref[...]

    lo = jnp.min(c, axis=1, keepdims=True)
    hi = jnp.max(c, axis=1, keepdims=True)

    def it(_, lh):
        lo_, hi_ = lh
        mid = (lo_ + hi_) * 0.5
        cnt = jnp.sum((c >= mid).astype(jnp.float32), axis=1, keepdims=True)
        ge = cnt >= _K
        return (jnp.where(ge, mid, lo_), jnp.where(ge, hi_, mid))

    lo, hi = jax.lax.fori_loop(0, n_iter, it, (lo, hi))
    masked = jnp.where(c >= lo, c, 0.0)
    mc_ref[...] = masked

    i = pl.program_id(0)

    @pl.when(i == 0)
    def _():
        sp_ref[...] = jnp.zeros((1, 1), jnp.float32)

    sp_ref[...] += jnp.sum(jnp.abs(masked)).reshape(1, 1)

    @pl.when(i == nsteps - 1)
    def _():
        sp_ref[...] = sp_ref[...] / denom


def _stageB_body(mc_ref, d_ref, basis_ref, bias_ref, g_ref, b_ref,
                 wq_ref, wk_ref, wv_ref,
                 q_ref, k_ref, v_ref, e_ref, acc_ref, *, nmt, nsteps, denom):
    i = pl.program_id(0)
    j = pl.program_id(1)

    @pl.when(j == 0)
    def _():
        acc_ref[...] = basis_ref[...] + bias_ref[...]

    acc_ref[...] += jnp.dot(mc_ref[...], d_ref[...])

    @pl.when(j == nmt - 1)
    def _():
        xr = acc_ref[...]

        @pl.when(i == 0)
        def _():
            e_ref[...] = jnp.zeros((1, 1), jnp.float32)

        e_ref[...] += jnp.sum(xr * xr).reshape(1, 1)

        @pl.when(i == nsteps - 1)
        def _():
            e_ref[...] = e_ref[...] / denom

        n = _ln(xr, g_ref[...], b_ref[...])
        q_ref[...] = jnp.dot(n, wq_ref[...])
        k_ref[...] = jnp.dot(n, wk_ref[...])
        v_ref[...] = jnp.dot(n, wv_ref[...])


def _attn_body(q_ref, k_ref, v_ref, w_ref, ctx_ref, kst_ref, vst_ref,
               *, scale, dh, t_len):
    i = pl.program_id(2)

    @pl.when(i == 0)
    def _():
        k2 = k_ref[0]
        v2 = v_ref[0]
        z = jnp.zeros((t_len, dh), jnp.float32)
        kst_ref[...] = jnp.concatenate([
            jnp.concatenate([k2[:, :dh], z], axis=1),
            jnp.concatenate([z, k2[:, dh:]], axis=1)], axis=0)
        vst_ref[...] = jnp.concatenate([
            jnp.concatenate([v2[:, :dh], z], axis=1),
            jnp.concatenate([z, v2[:, dh:]], axis=1)], axis=0)

    q2 = q_ref[0]
    s_cat = jax.lax.dot_general(q2, kst_ref[...], (((1,), (1,)), ((), ()))) * scale
    w0 = _softmax_last(s_cat[:, :t_len])
    w1 = _softmax_last(s_cat[:, t_len:])
    w_ref[0, 0] = w0
    w_ref[0, 1] = w1
    w_cat = jnp.concatenate([w0, w1], axis=1)
    ctx_ref[0] = jnp.dot(w_cat, vst_ref[...])


def _stageE_body(c_ref, wo_ref, x_ref, g_ref, b_ref, x1_ref, n2_ref, e_ref,
                 *, nsteps, denom):
    cb = c_ref[...].astype(jnp.bfloat16)
    x1 = jnp.dot(cb, wo_ref[...], preferred_element_type=jnp.float32) + x_ref[...]
    x1_ref[...] = x1
    i = pl.program_id(0)

    @pl.when(i == 0)
    def _():
        e_ref[...] = jnp.zeros((1, 1), jnp.float32)

    e_ref[...] += jnp.sum(x1 * x1).reshape(1, 1)

    @pl.when(i == nsteps - 1)
    def _():
        e_ref[...] = e_ref[...] / denom

    n2_ref[...] = _ln(x1, g_ref[...], b_ref[...])


def _ffn_body(n_ref, wg_ref, w1_ref, w2_ref, x_ref, o_ref):
    j = pl.program_id(1)
    nb = n_ref[...].astype(jnp.bfloat16)
    f32 = jnp.float32
    h = (jax.nn.sigmoid(jnp.dot(nb, wg_ref[...], preferred_element_type=f32))
         * jax.nn.gelu(jnp.dot(nb, w1_ref[...], preferred_element_type=f32)))

    @pl.when(j == 0)
    def _():
        o_ref[...] = x_ref[...]

    o_ref[...] += jnp.dot(h.astype(jnp.bfloat16), w2_ref[...],
                          preferred_element_type=f32)


def kernel(x, W_fam, fam_emb, W_enc, b_enc, dictionary, bias_correction,
           ln1_g, ln1_b, Wq, Wk, Wv, Wo, ln2_g, ln2_b, W1, Wg, W2):
    B, T, D = x.shape
    F = W_fam.shape[1]
    M = W_enc.shape[1]
    DFF = W1.shape[1]
    N = B * T
    H = _H
    dh = D // H

    R = 256
    nt = N // R
    MT = 512
    nmt = M // MT
    FT = 512
    nft = DFF // FT
    TQ = 256
    f32 = jnp.float32

    xf = x.reshape(N, D)

    fs, basis, masked, sp = pl.pallas_call(
        functools.partial(_stageA_body, nsteps=nt, denom=float(N * M), n_iter=28),
        grid=(nt,),
        in_specs=[
            pl.BlockSpec((R, D), lambda i: (i, 0)),
            pl.BlockSpec((D, F), lambda i: (0, 0)),
            pl.BlockSpec((F, D), lambda i: (0, 0)),
            pl.BlockSpec((D, M), lambda i: (0, 0)),
            pl.BlockSpec((1, M), lambda i: (0, 0)),
        ],
        out_specs=[
            pl.BlockSpec((R, F), lambda i: (i, 0)),
            pl.BlockSpec((R, D), lambda i: (i, 0)),
            pl.BlockSpec((R, M), lambda i: (i, 0)),
            pl.BlockSpec((1, 1), lambda i: (0, 0)),
        ],
        out_shape=[
            jax.ShapeDtypeStruct((N, F), f32),
            jax.ShapeDtypeStruct((N, D), f32),
            jax.ShapeDtypeStruct((N, M), f32),
            jax.ShapeDtypeStruct((1, 1), f32),
        ],
    )(xf, W_fam, fam_emb, W_enc, b_enc.reshape(1, M))

    qf, kf, vf, e1 = pl.pallas_call(
        functools.partial(_stageB_body, nmt=nmt, nsteps=nt, denom=float(N * D)),
        grid=(nt, nmt),
        in_specs=[
            pl.BlockSpec((R, MT), lambda i, j: (i, j)),
            pl.BlockSpec((MT, D), lambda i, j: (j, 0)),
            pl.BlockSpec((R, D), lambda i, j: (i, 0)),
            pl.BlockSpec((1, D), lambda i, j: (0, 0)),
            pl.BlockSpec((1, D), lambda i, j: (0, 0)),
            pl.BlockSpec((1, D), lambda i, j: (0, 0)),
            pl.BlockSpec((D, D), lambda i, j: (0, 0)),
            pl.BlockSpec((D, D), lambda i, j: (0, 0)),
            pl.BlockSpec((D, D), lambda i, j: (0, 0)),
        ],
        out_specs=[
            pl.BlockSpec((R, D), lambda i, j: (i, 0)),
            pl.BlockSpec((R, D), lambda i, j: (i, 0)),
            pl.BlockSpec((R, D), lambda i, j: (i, 0)),
            pl.BlockSpec((1, 1), lambda i, j: (0, 0)),
        ],
        out_shape=[
            jax.ShapeDtypeStruct((N, D), f32),
            jax.ShapeDtypeStruct((N, D), f32),
            jax.ShapeDtypeStruct((N, D), f32),
            jax.ShapeDtypeStruct((1, 1), f32),
        ],
        scratch_shapes=[pltpu.VMEM((R, D), f32)],
    )(masked, dictionary, basis, bias_correction.reshape(1, D),
      ln1_g.reshape(1, D), ln1_b.reshape(1, D), Wq, Wk, Wv)

    q3 = qf.reshape(B, T, D)
    k3 = kf.reshape(B, T, D)
    v3 = vf.reshape(B, T, D)

    attn_w, ctx3 = pl.pallas_call(
        functools.partial(_attn_body, scale=1.0 / float(dh) ** 0.5, dh=dh, t_len=T),
        grid=(B, H // 2, T // TQ),
        in_specs=[
            pl.BlockSpec((1, TQ, 2 * dh), lambda b, hp, i: (b, i, hp)),
            pl.BlockSpec((1, T, 2 * dh), lambda b, hp, i: (b, 0, hp)),
            pl.BlockSpec((1, T, 2 * dh), lambda b, hp, i: (b, 0, hp)),
        ],
        out_specs=[
            pl.BlockSpec((1, 2, TQ, T), lambda b, hp, i: (b, hp, i, 0)),
            pl.BlockSpec((1, TQ, 2 * dh), lambda b, hp, i: (b, i, hp)),
        ],
        out_shape=[
            jax.ShapeDtypeStruct((B, H, T, T), f32),
            jax.ShapeDtypeStruct((B, T, D), f32),
        ],
        scratch_shapes=[
            pltpu.VMEM((2 * T, 2 * dh), f32),
            pltpu.VMEM((2 * T, 2 * dh), f32),
        ],
    )(q3, k3, v3)

    ctx = ctx3.reshape(N, D)

    x1, normed2, e2 = pl.pallas_call(
        functools.partial(_stageE_body, nsteps=nt, denom=float(N * D)),
        grid=(nt,),
        in_specs=[
            pl.BlockSpec((R, D), lambda i: (i, 0)),
            pl.BlockSpec((D, D), lambda i: (0, 0)),
            pl.BlockSpec((R, D), lambda i: (i, 0)),
            pl.BlockSpec((1, D), lambda i: (0, 0)),
            pl.BlockSpec((1, D), lambda i: (0, 0)),
        ],
        out_specs=[
            pl.BlockSpec((R, D), lambda i: (i, 0)),
            pl.BlockSpec((R, D), lambda i: (i, 0)),
            pl.BlockSpec((1, 1), lambda i: (0, 0)),
        ],
        out_shape=[
            jax.ShapeDtypeStruct((N, D), f32),
            jax.ShapeDtypeStruct((N, D), f32),
            jax.ShapeDtypeStruct((1, 1), f32),
        ],
    )(ctx, Wo.astype(jnp.bfloat16), xf, ln2_g.reshape(1, D), ln2_b.reshape(1, D))

    xout = pl.pallas_call(
        _ffn_body,
        grid=(nt, nft),
        in_specs=[
            pl.BlockSpec((R, D), lambda i, j: (i, 0)),
            pl.BlockSpec((D, FT), lambda i, j: (0, j)),
            pl.BlockSpec((D, FT), lambda i, j: (0, j)),
            pl.BlockSpec((FT, D), lambda i, j: (j, 0)),
            pl.BlockSpec((R, D), lambda i, j: (i, 0)),
        ],
        out_specs=pl.BlockSpec((R, D), lambda i, j: (i, 0)),
        out_shape=jax.ShapeDtypeStruct((N, D), f32),
    )(normed2, Wg.astype(jnp.bfloat16), W1.astype(jnp.bfloat16),
      W2.astype(jnp.bfloat16), x1)

    return (
        xout.reshape(B, T, D),
        attn_w,
        fs.reshape(B, T, F),
        sp.reshape(()),
        e1.reshape(()),
        e2.reshape(()),
    )
